# baseline (speedup 1.0000x reference)
import functools

import jax
import jax.numpy as jnp
from jax import lax
from jax.experimental import pallas as pl
from jax.experimental.pallas import tpu as pltpu
from jax.experimental.pallas import tpu_sc as plsc

_QB = 512
_MB = 4096
_GR = 32
_LN = 128
_BIGF = 2.0 ** 26
_TOPK = 3


def _phase1_body(q_ref, mem_ref, i1_ref, i2_ref, i3_ref,
                 sims_a, sims_b, tv1, tv2, tv3, ti1, ti2, ti3,
                 v1_ref, v2_ref, v3_ref, g1_ref, g2_ref, g3_ref,
                 *, total_rows, mb, num_m, num_q):
    mi = pl.program_id(0)
    qi = pl.program_id(1)
    qb = _QB
    s = mi * num_q + qi
    neginf = jnp.float32(-jnp.inf)

    @pl.when(s == 0)
    def _init():
        neg = jnp.full((num_q * qb, 1), neginf, jnp.float32)
        zero = jnp.zeros((num_q * qb, 1), jnp.int32)
        v1_ref[...] = neg
        v2_ref[...] = neg
        v3_ref[...] = neg
        g1_ref[...] = zero
        g2_ref[...] = zero
        g3_ref[...] = zero

    def _step(sims_ref, write_ref):
        m_prev = (s - 1) // num_q
        q_prev = (s - 1) % num_q
        nch = mb // _LN
        ngr = qb // _GR
        msl = mb // ngr

        def group_body(g, carry):
            write_ref[:, pl.ds(g * msl, msl)] = lax.dot_general(
                q_ref[...], mem_ref[pl.ds(g * msl, msl), :],
                dimension_numbers=(((1,), (1,)), ((), ())),
                preferred_element_type=jnp.float32)
            rows_g = pl.ds(g * _GR, _GR)
            v1 = jnp.full((_GR, _LN), neginf, jnp.float32)
            v2 = v1
            v3 = v1
            i1 = jnp.zeros((_GR, _LN), jnp.float32)
            i2 = i1
            i3 = i1
            lane = lax.broadcasted_iota(jnp.int32, (_GR, _LN), 1).astype(jnp.float32)
            for c in range(nch):
                ic = lane + jnp.float32(c * _LN)
                x = sims_ref[rows_g, c * _LN:(c + 1) * _LN]
                gt1 = x > v1
                gt2 = x > v2
                gt3 = x > v3
                nv1 = jnp.where(gt1, x, v1)
                ni1 = jnp.where(gt1, ic, i1)
                nv2 = jnp.where(gt1, v1, jnp.where(gt2, x, v2))
                ni2 = jnp.where(gt1, i1, jnp.where(gt2, ic, i2))
                nv3 = jnp.where(gt2, v2, jnp.where(gt3, x, v3))
                ni3 = jnp.where(gt2, i2, jnp.where(gt3, ic, i3))
                v1, v2, v3 = nv1, nv2, nv3
                i1, i2, i3 = ni1, ni2, ni3
            tv1[rows_g, :] = v1
            tv2[rows_g, :] = v2
            tv3[rows_g, :] = v3
            ti1[rows_g, :] = i1
            ti2[rows_g, :] = i2
            ti3[rows_g, :] = i3
            return carry

        lax.fori_loop(0, ngr, group_body, jnp.int32(0))

        @pl.when(mi == num_m - 1)
        def _mask_tail():
            col = lax.broadcasted_iota(jnp.int32, (qb, mb), 1)
            limit = total_rows - mi * mb
            write_ref[...] = jnp.where(col < limit, write_ref[...], neginf)

        V = jnp.concatenate([tv1[...], tv2[...], tv3[...]], axis=1)
        I = jnp.concatenate([ti1[...], ti2[...], ti3[...]], axis=1)
        cand = []
        for t in range(_TOPK):
            m = jnp.max(V, axis=1, keepdims=True)
            eq = V == m
            it = jnp.min(jnp.where(eq, I, _BIGF), axis=1, keepdims=True)
            gc = (it + (m_prev * mb).astype(jnp.float32)).astype(jnp.int32)
            cand.append((jnp.where((gc >= 0) & (gc < total_rows), m, neginf),
                         gc))
            if t < _TOPK - 1:
                V = jnp.where(eq & (I == it), neginf, V)

        rows = pl.ds(q_prev * qb, qb)
        rv1 = v1_ref[rows, :]
        rv2 = v2_ref[rows, :]
        rv3 = v3_ref[rows, :]
        ri1 = g1_ref[rows, :]
        ri2 = g2_ref[rows, :]
        ri3 = g3_ref[rows, :]
        for cv, gc in cand:
            gt1 = cv > rv1
            gt2 = cv > rv2
            gt3 = cv > rv3
            nv1 = jnp.where(gt1, cv, rv1)
            ni1 = jnp.where(gt1, gc, ri1)
            nv2 = jnp.where(gt1, rv1, jnp.where(gt2, cv, rv2))
            ni2 = jnp.where(gt1, ri1, jnp.where(gt2, gc, ri2))
            nv3 = jnp.where(gt2, rv2, jnp.where(gt3, cv, rv3))
            ni3 = jnp.where(gt2, ri2, jnp.where(gt3, gc, ri3))
            rv1, rv2, rv3 = nv1, nv2, nv3
            ri1, ri2, ri3 = ni1, ni2, ni3
        v1_ref[rows, :] = rv1
        v2_ref[rows, :] = rv2
        v3_ref[rows, :] = rv3
        g1_ref[rows, :] = ri1
        g2_ref[rows, :] = ri2
        g3_ref[rows, :] = ri3
        i1_ref[...] = ri1
        i2_ref[...] = ri2
        i3_ref[...] = ri3

    @pl.when((s % 2) == 0)
    def _step_even():
        _step(sims_b, sims_a)

    @pl.when((s % 2) == 1)
    def _step_odd():
        _step(sims_a, sims_b)


def _phase1_topk(query, memory):
    b, d = query.shape
    total_rows = memory.shape[0]
    num_q = b // _QB
    num_m = pl.cdiv(total_rows, _MB)
    body = functools.partial(_phase1_body, total_rows=total_rows, mb=_MB,
                             num_m=num_m, num_q=num_q)
    out = pl.pallas_call(
        body,
        grid=(num_m + 1, num_q),
        in_specs=[
            pl.BlockSpec((_QB, d), lambda mi, qi: (qi, 0)),
            pl.BlockSpec((_MB, d),
                         lambda mi, qi, _nm=num_m: (jnp.minimum(mi, _nm - 1), 0)),
        ],
        out_specs=[pl.BlockSpec(
            (_QB, 1),
            lambda mi, qi, _nq=num_q: ((qi + _nq - 1) % _nq, 0))] * 3,
        out_shape=[jax.ShapeDtypeStruct((b, 1), jnp.int32)] * 3,
        scratch_shapes=(
            [pltpu.VMEM((_QB, _MB), jnp.float32)] * 2
            + [pltpu.VMEM((_QB, _LN), jnp.float32)] * 6
            + [pltpu.VMEM((b, 1), jnp.float32)] * 3
            + [pltpu.VMEM((b, 1), jnp.int32)] * 3
        ),
    )(query, memory)
    return out


def _phase2_gather_sum(idx_all, memory):
    b = idx_all.shape[0] // _TOPK
    d = memory.shape[1]
    info = plsc.get_sparse_core_info()
    nc, ns = info.num_cores, info.num_subcores
    nw = nc * ns
    bw = b // nw
    nch = d // 16
    mesh = plsc.VectorSubcoreMesh(core_axis_name="c", subcore_axis_name="s")

    @functools.partial(
        pl.kernel,
        mesh=mesh,
        out_type=jax.ShapeDtypeStruct((nw, _TOPK, d), jnp.float32),
        scratch_types=[
            pltpu.VMEM((bw,), jnp.int32),
            pltpu.VMEM((bw, d), jnp.float32),
            pltpu.VMEM((_TOPK, d), jnp.float32),
            pltpu.SemaphoreType.DMA,
        ],
    )
    def sc_kernel(idx_hbm, mem_hbm, out_hbm, idx_v, rows_v, acc_v, sem):
        wid = lax.axis_index("s") * nc + lax.axis_index("c")
        base = wid * bw
        for j in range(_TOPK):
            pltpu.sync_copy(idx_hbm.at[pl.ds(j * b + base, bw)], idx_v)
            pltpu.async_copy(mem_hbm.at[idx_v], rows_v, sem).wait()

            def acc_body(r, carry):
                return tuple(carry[c] + rows_v[r, pl.ds(c * 16, 16)]
                             for c in range(nch))

            accs = lax.fori_loop(
                0, bw, acc_body,
                tuple(jnp.zeros((16,), jnp.float32) for _ in range(nch)))
            for c in range(nch):
                acc_v[j, pl.ds(c * 16, 16)] = accs[c]
        pltpu.sync_copy(acc_v, out_hbm.at[wid])

    return sc_kernel(idx_all, memory)


def _phase3_reduce(partials, scale):
    nw, flat = partials.shape

    def body(p_ref, o_ref):
        o_ref[...] = jnp.sum(p_ref[...], axis=0, keepdims=True) * scale

    return pl.pallas_call(
        body,
        out_shape=jax.ShapeDtypeStruct((1, flat), jnp.float32),
    )(partials)


def kernel(query, episodic_memory):
    b, d = query.shape
    i1, i2, i3 = _phase1_topk(query, episodic_memory)
    idx_all = jnp.concatenate([i1[:, 0], i2[:, 0], i3[:, 0]])
    partials = _phase2_gather_sum(idx_all, episodic_memory)
    out = _phase3_reduce(partials.reshape(-1, _TOPK * d), 1.0 / b)
    return out.reshape(_TOPK, d)


# speedup vs baseline: 1.4094x; 1.4094x over previous
import functools

import jax
import jax.numpy as jnp
from jax import lax
from jax.experimental import pallas as pl
from jax.experimental.pallas import tpu as pltpu
from jax.experimental.pallas import tpu_sc as plsc

_QB = 512
_MB = 4096
_GR = 32
_LN = 128
_BIGF = 2.0 ** 26
_TOPK = 3


def _phase1_body(q_ref, mem_ref, i1_ref, i2_ref, i3_ref,
                 sims_a, sims_b, tv1, tv2, tv3, ti1, ti2, ti3,
                 v1_ref, v2_ref, v3_ref, g1_ref, g2_ref, g3_ref,
                 *, total_rows, mb, num_m, num_q):
    mi = pl.program_id(0)
    qi = pl.program_id(1)
    qb = _QB
    s = mi * num_q + qi
    neginf = jnp.float32(-jnp.inf)

    @pl.when(s == 0)
    def _init():
        neg = jnp.full((num_q * qb, 1), neginf, jnp.float32)
        zero = jnp.zeros((num_q * qb, 1), jnp.int32)
        v1_ref[...] = neg
        v2_ref[...] = neg
        v3_ref[...] = neg
        g1_ref[...] = zero
        g2_ref[...] = zero
        g3_ref[...] = zero

    def _step(sims_ref, write_ref):
        m_prev = (s - 1) // num_q
        q_prev = (s - 1) % num_q
        nch = mb // _LN
        ngr = qb // _GR
        nouter = 4
        ginner = ngr // nouter
        msl = mb // nouter

        def outer_body(o, carry):
            write_ref[:, pl.ds(o * msl, msl)] = lax.dot_general(
                q_ref[...], mem_ref[pl.ds(o * msl, msl), :],
                dimension_numbers=(((1,), (1,)), ((), ())),
                preferred_element_type=jnp.float32)
            lane = lax.broadcasted_iota(jnp.int32, (_GR, _LN), 1).astype(jnp.float32)
            for sub in range(ginner):
                rows_g = pl.ds((o * ginner + sub) * _GR, _GR)
                v1 = jnp.full((_GR, _LN), neginf, jnp.float32)
                v2 = v1
                v3 = v1
                i1 = jnp.zeros((_GR, _LN), jnp.float32)
                i2 = i1
                i3 = i1
                for c in range(nch):
                    ic = lane + jnp.float32(c * _LN)
                    x = sims_ref[rows_g, c * _LN:(c + 1) * _LN]
                    gt1 = x > v1
                    gt2 = x > v2
                    gt3 = x > v3
                    nv1 = jnp.where(gt1, x, v1)
                    ni1 = jnp.where(gt1, ic, i1)
                    nv2 = jnp.where(gt1, v1, jnp.where(gt2, x, v2))
                    ni2 = jnp.where(gt1, i1, jnp.where(gt2, ic, i2))
                    nv3 = jnp.where(gt2, v2, jnp.where(gt3, x, v3))
                    ni3 = jnp.where(gt2, i2, jnp.where(gt3, ic, i3))
                    v1, v2, v3 = nv1, nv2, nv3
                    i1, i2, i3 = ni1, ni2, ni3
                tv1[rows_g, :] = v1
                tv2[rows_g, :] = v2
                tv3[rows_g, :] = v3
                ti1[rows_g, :] = i1
                ti2[rows_g, :] = i2
                ti3[rows_g, :] = i3
            return carry

        lax.fori_loop(0, nouter, outer_body, jnp.int32(0))

        @pl.when(mi == num_m - 1)
        def _mask_tail():
            col = lax.broadcasted_iota(jnp.int32, (qb, mb), 1)
            limit = total_rows - mi * mb
            write_ref[...] = jnp.where(col < limit, write_ref[...], neginf)

        V = jnp.concatenate([tv1[...], tv2[...], tv3[...]], axis=1)
        I = jnp.concatenate([ti1[...], ti2[...], ti3[...]], axis=1)
        cand = []
        for t in range(_TOPK):
            m = jnp.max(V, axis=1, keepdims=True)
            eq = V == m
            it = jnp.min(jnp.where(eq, I, _BIGF), axis=1, keepdims=True)
            gc = (it + (m_prev * mb).astype(jnp.float32)).astype(jnp.int32)
            cand.append((jnp.where((gc >= 0) & (gc < total_rows), m, neginf),
                         gc))
            if t < _TOPK - 1:
                V = jnp.where(eq & (I == it), neginf, V)

        rows = pl.ds(q_prev * qb, qb)
        rv1 = v1_ref[rows, :]
        rv2 = v2_ref[rows, :]
        rv3 = v3_ref[rows, :]
        ri1 = g1_ref[rows, :]
        ri2 = g2_ref[rows, :]
        ri3 = g3_ref[rows, :]
        for cv, gc in cand:
            gt1 = cv > rv1
            gt2 = cv > rv2
            gt3 = cv > rv3
            nv1 = jnp.where(gt1, cv, rv1)
            ni1 = jnp.where(gt1, gc, ri1)
            nv2 = jnp.where(gt1, rv1, jnp.where(gt2, cv, rv2))
            ni2 = jnp.where(gt1, ri1, jnp.where(gt2, gc, ri2))
            nv3 = jnp.where(gt2, rv2, jnp.where(gt3, cv, rv3))
            ni3 = jnp.where(gt2, ri2, jnp.where(gt3, gc, ri3))
            rv1, rv2, rv3 = nv1, nv2, nv3
            ri1, ri2, ri3 = ni1, ni2, ni3
        v1_ref[rows, :] = rv1
        v2_ref[rows, :] = rv2
        v3_ref[rows, :] = rv3
        g1_ref[rows, :] = ri1
        g2_ref[rows, :] = ri2
        g3_ref[rows, :] = ri3
        i1_ref[...] = ri1
        i2_ref[...] = ri2
        i3_ref[...] = ri3

    @pl.when((s % 2) == 0)
    def _step_even():
        _step(sims_b, sims_a)

    @pl.when((s % 2) == 1)
    def _step_odd():
        _step(sims_a, sims_b)


def _phase1_topk(query, memory):
    b, d = query.shape
    total_rows = memory.shape[0]
    num_q = b // _QB
    num_m = pl.cdiv(total_rows, _MB)
    body = functools.partial(_phase1_body, total_rows=total_rows, mb=_MB,
                             num_m=num_m, num_q=num_q)
    out = pl.pallas_call(
        body,
        grid=(num_m + 1, num_q),
        in_specs=[
            pl.BlockSpec((_QB, d), lambda mi, qi: (qi, 0)),
            pl.BlockSpec((_MB, d),
                         lambda mi, qi, _nm=num_m: (jnp.minimum(mi, _nm - 1), 0)),
        ],
        out_specs=[pl.BlockSpec(
            (_QB, 1),
            lambda mi, qi, _nq=num_q: ((qi + _nq - 1) % _nq, 0))] * 3,
        out_shape=[jax.ShapeDtypeStruct((b, 1), jnp.int32)] * 3,
        scratch_shapes=(
            [pltpu.VMEM((_QB, _MB), jnp.float32)] * 2
            + [pltpu.VMEM((_QB, _LN), jnp.float32)] * 6
            + [pltpu.VMEM((b, 1), jnp.float32)] * 3
            + [pltpu.VMEM((b, 1), jnp.int32)] * 3
        ),
    )(query, memory)
    return out


def _phase2_gather_sum(idx_all, memory):
    b = idx_all.shape[0] // _TOPK
    d = memory.shape[1]
    info = plsc.get_sparse_core_info()
    nc, ns = info.num_cores, info.num_subcores
    nw = nc * ns
    bw = b // nw
    nch = d // 16
    mesh = plsc.VectorSubcoreMesh(core_axis_name="c", subcore_axis_name="s")

    @functools.partial(
        pl.kernel,
        mesh=mesh,
        out_type=jax.ShapeDtypeStruct((nw, _TOPK, d), jnp.float32),
        scratch_types=[
            pltpu.VMEM((bw,), jnp.int32),
            pltpu.VMEM((bw, d), jnp.float32),
            pltpu.VMEM((_TOPK, d), jnp.float32),
            pltpu.SemaphoreType.DMA,
        ],
    )
    def sc_kernel(idx_hbm, mem_hbm, out_hbm, idx_v, rows_v, acc_v, sem):
        wid = lax.axis_index("s") * nc + lax.axis_index("c")
        base = wid * bw
        for j in range(_TOPK):
            pltpu.sync_copy(idx_hbm.at[pl.ds(j * b + base, bw)], idx_v)
            pltpu.async_copy(mem_hbm.at[idx_v], rows_v, sem).wait()

            def acc_body(r, carry):
                return tuple(carry[c] + rows_v[r, pl.ds(c * 16, 16)]
                             for c in range(nch))

            accs = lax.fori_loop(
                0, bw, acc_body,
                tuple(jnp.zeros((16,), jnp.float32) for _ in range(nch)))
            for c in range(nch):
                acc_v[j, pl.ds(c * 16, 16)] = accs[c]
        pltpu.sync_copy(acc_v, out_hbm.at[wid])

    return sc_kernel(idx_all, memory)


def _phase3_reduce(partials, scale):
    nw, flat = partials.shape

    def body(p_ref, o_ref):
        o_ref[...] = jnp.sum(p_ref[...], axis=0, keepdims=True) * scale

    return pl.pallas_call(
        body,
        out_shape=jax.ShapeDtypeStruct((1, flat), jnp.float32),
    )(partials)


def kernel(query, episodic_memory):
    b, d = query.shape
    i1, i2, i3 = _phase1_topk(query, episodic_memory)
    idx_all = jnp.concatenate([i1[:, 0], i2[:, 0], i3[:, 0]])
    partials = _phase2_gather_sum(idx_all, episodic_memory)
    out = _phase3_reduce(partials.reshape(-1, _TOPK * d), 1.0 / b)
    return out.reshape(_TOPK, d)


# speedup vs baseline: 1.6132x; 1.1446x over previous
import functools

import jax
import jax.numpy as jnp
from jax import lax
from jax.experimental import pallas as pl
from jax.experimental.pallas import tpu as pltpu
from jax.experimental.pallas import tpu_sc as plsc

_QB = 512
_MB = 4096
_GR = 32
_LN = 128
_BIGF = 2.0 ** 26
_TOPK = 3


def _phase1_body(q_ref, mem_ref, i1_ref, i2_ref, i3_ref,
                 sims_a, sims_b, tv1, tv2, tv3, ti1, ti2, ti3,
                 v1_ref, v2_ref, v3_ref, g1_ref, g2_ref, g3_ref,
                 *, total_rows, mb, num_m, num_q):
    mi = pl.program_id(0)
    qi = pl.program_id(1)
    qb = _QB
    s = mi * num_q + qi
    neginf = jnp.float32(-jnp.inf)

    @pl.when(s == 0)
    def _init():
        neg = jnp.full((num_q * qb, 1), neginf, jnp.float32)
        zero = jnp.zeros((num_q * qb, 1), jnp.int32)
        v1_ref[...] = neg
        v2_ref[...] = neg
        v3_ref[...] = neg
        g1_ref[...] = zero
        g2_ref[...] = zero
        g3_ref[...] = zero

    def _step(sims_ref, write_ref):
        m_prev = (s - 1) // num_q
        q_prev = (s - 1) % num_q
        nch = mb // _LN
        ngr = qb // _GR
        nouter = 4
        ginner = ngr // nouter
        msl = mb // nouter

        lane = lax.broadcasted_iota(jnp.int32, (_GR, _LN), 1).astype(jnp.float32)
        for o in range(nouter):
            write_ref[:, o * msl:(o + 1) * msl] = lax.dot_general(
                q_ref[...], mem_ref[o * msl:(o + 1) * msl, :],
                dimension_numbers=(((1,), (1,)), ((), ())),
                preferred_element_type=jnp.float32)
            for sub in range(ginner):
                rows_g = pl.ds((o * ginner + sub) * _GR, _GR)
                v1 = jnp.full((_GR, _LN), neginf, jnp.float32)
                v2 = v1
                v3 = v1
                i1 = jnp.zeros((_GR, _LN), jnp.float32)
                i2 = i1
                i3 = i1
                for c in range(nch):
                    ic = lane + jnp.float32(c * _LN)
                    x = sims_ref[rows_g, c * _LN:(c + 1) * _LN]
                    gt1 = x > v1
                    gt2 = x > v2
                    gt3 = x > v3
                    nv1 = jnp.where(gt1, x, v1)
                    ni1 = jnp.where(gt1, ic, i1)
                    nv2 = jnp.where(gt1, v1, jnp.where(gt2, x, v2))
                    ni2 = jnp.where(gt1, i1, jnp.where(gt2, ic, i2))
                    nv3 = jnp.where(gt2, v2, jnp.where(gt3, x, v3))
                    ni3 = jnp.where(gt2, i2, jnp.where(gt3, ic, i3))
                    v1, v2, v3 = nv1, nv2, nv3
                    i1, i2, i3 = ni1, ni2, ni3
                tv1[rows_g, :] = v1
                tv2[rows_g, :] = v2
                tv3[rows_g, :] = v3
                ti1[rows_g, :] = i1
                ti2[rows_g, :] = i2
                ti3[rows_g, :] = i3

        @pl.when(mi == num_m - 1)
        def _mask_tail():
            col = lax.broadcasted_iota(jnp.int32, (qb, mb), 1)
            limit = total_rows - mi * mb
            write_ref[...] = jnp.where(col < limit, write_ref[...], neginf)

        V = jnp.concatenate([tv1[...], tv2[...], tv3[...]], axis=1)
        I = jnp.concatenate([ti1[...], ti2[...], ti3[...]], axis=1)
        cand = []
        for t in range(_TOPK):
            m = jnp.max(V, axis=1, keepdims=True)
            eq = V == m
            it = jnp.min(jnp.where(eq, I, _BIGF), axis=1, keepdims=True)
            gc = (it + (m_prev * mb).astype(jnp.float32)).astype(jnp.int32)
            cand.append((jnp.where((gc >= 0) & (gc < total_rows), m, neginf),
                         gc))
            if t < _TOPK - 1:
                V = jnp.where(eq & (I == it), neginf, V)

        rows = pl.ds(q_prev * qb, qb)
        rv1 = v1_ref[rows, :]
        rv2 = v2_ref[rows, :]
        rv3 = v3_ref[rows, :]
        ri1 = g1_ref[rows, :]
        ri2 = g2_ref[rows, :]
        ri3 = g3_ref[rows, :]
        for cv, gc in cand:
            gt1 = cv > rv1
            gt2 = cv > rv2
            gt3 = cv > rv3
            nv1 = jnp.where(gt1, cv, rv1)
            ni1 = jnp.where(gt1, gc, ri1)
            nv2 = jnp.where(gt1, rv1, jnp.where(gt2, cv, rv2))
            ni2 = jnp.where(gt1, ri1, jnp.where(gt2, gc, ri2))
            nv3 = jnp.where(gt2, rv2, jnp.where(gt3, cv, rv3))
            ni3 = jnp.where(gt2, ri2, jnp.where(gt3, gc, ri3))
            rv1, rv2, rv3 = nv1, nv2, nv3
            ri1, ri2, ri3 = ni1, ni2, ni3
        v1_ref[rows, :] = rv1
        v2_ref[rows, :] = rv2
        v3_ref[rows, :] = rv3
        g1_ref[rows, :] = ri1
        g2_ref[rows, :] = ri2
        g3_ref[rows, :] = ri3
        i1_ref[...] = ri1
        i2_ref[...] = ri2
        i3_ref[...] = ri3

    @pl.when((s % 2) == 0)
    def _step_even():
        _step(sims_b, sims_a)

    @pl.when((s % 2) == 1)
    def _step_odd():
        _step(sims_a, sims_b)


def _phase1_topk(query, memory):
    b, d = query.shape
    total_rows = memory.shape[0]
    num_q = b // _QB
    num_m = pl.cdiv(total_rows, _MB)
    body = functools.partial(_phase1_body, total_rows=total_rows, mb=_MB,
                             num_m=num_m, num_q=num_q)
    out = pl.pallas_call(
        body,
        grid=(num_m + 1, num_q),
        in_specs=[
            pl.BlockSpec((_QB, d), lambda mi, qi: (qi, 0)),
            pl.BlockSpec((_MB, d),
                         lambda mi, qi, _nm=num_m: (jnp.minimum(mi, _nm - 1), 0)),
        ],
        out_specs=[pl.BlockSpec(
            (_QB, 1),
            lambda mi, qi, _nq=num_q: ((qi + _nq - 1) % _nq, 0))] * 3,
        out_shape=[jax.ShapeDtypeStruct((b, 1), jnp.int32)] * 3,
        scratch_shapes=(
            [pltpu.VMEM((_QB, _MB), jnp.float32)] * 2
            + [pltpu.VMEM((_QB, _LN), jnp.float32)] * 6
            + [pltpu.VMEM((b, 1), jnp.float32)] * 3
            + [pltpu.VMEM((b, 1), jnp.int32)] * 3
        ),
    )(query, memory)
    return out


def _phase2_gather_sum(idx_all, memory):
    b = idx_all.shape[0] // _TOPK
    d = memory.shape[1]
    info = plsc.get_sparse_core_info()
    nc, ns = info.num_cores, info.num_subcores
    nw = nc * ns
    bw = b // nw
    nch = d // 16
    mesh = plsc.VectorSubcoreMesh(core_axis_name="c", subcore_axis_name="s")

    @functools.partial(
        pl.kernel,
        mesh=mesh,
        out_type=jax.ShapeDtypeStruct((nw, _TOPK, d), jnp.float32),
        scratch_types=[
            pltpu.VMEM((bw,), jnp.int32),
            pltpu.VMEM((bw, d), jnp.float32),
            pltpu.VMEM((_TOPK, d), jnp.float32),
            pltpu.SemaphoreType.DMA,
        ],
    )
    def sc_kernel(idx_hbm, mem_hbm, out_hbm, idx_v, rows_v, acc_v, sem):
        wid = lax.axis_index("s") * nc + lax.axis_index("c")
        base = wid * bw
        for j in range(_TOPK):
            pltpu.sync_copy(idx_hbm.at[pl.ds(j * b + base, bw)], idx_v)
            pltpu.async_copy(mem_hbm.at[idx_v], rows_v, sem).wait()

            def acc_body(r, carry):
                return tuple(carry[c] + rows_v[r, pl.ds(c * 16, 16)]
                             for c in range(nch))

            accs = lax.fori_loop(
                0, bw, acc_body,
                tuple(jnp.zeros((16,), jnp.float32) for _ in range(nch)))
            for c in range(nch):
                acc_v[j, pl.ds(c * 16, 16)] = accs[c]
        pltpu.sync_copy(acc_v, out_hbm.at[wid])

    return sc_kernel(idx_all, memory)


def _phase3_reduce(partials, scale):
    nw, flat = partials.shape

    def body(p_ref, o_ref):
        o_ref[...] = jnp.sum(p_ref[...], axis=0, keepdims=True) * scale

    return pl.pallas_call(
        body,
        out_shape=jax.ShapeDtypeStruct((1, flat), jnp.float32),
    )(partials)


def kernel(query, episodic_memory):
    b, d = query.shape
    i1, i2, i3 = _phase1_topk(query, episodic_memory)
    idx_all = jnp.concatenate([i1[:, 0], i2[:, 0], i3[:, 0]])
    partials = _phase2_gather_sum(idx_all, episodic_memory)
    out = _phase3_reduce(partials.reshape(-1, _TOPK * d), 1.0 / b)
    return out.reshape(_TOPK, d)
